# baseline (speedup 1.0000x reference)
import functools

import jax
import jax.numpy as jnp
from jax.experimental import pallas as pl
from jax.experimental.pallas import tpu as pltpu

C = 128
L = 3
EPS = 0.0
TRI_TYPES = [(0, 1, 1), (1, 1, 1), (1, 1, 2), (1, 2, 2), (2, 2, 2), (3, 2, 1), (3, 3, 1)]


def _matmul_bias_kernel(x_ref, w_ref, b_ref, o_ref):
    o_ref[...] = (
        jnp.dot(x_ref[...], w_ref[...], preferred_element_type=jnp.float32)
        + b_ref[...]
    )


def _matmul_bias(x, w, b, block=1024):
    m = x.shape[0]
    pad = (-m) % block
    xp = jnp.pad(x, ((0, pad), (0, 0))) if pad else x
    mp = m + pad
    out = pl.pallas_call(
        _matmul_bias_kernel,
        grid=(mp // block,),
        in_specs=[
            pl.BlockSpec((block, C), lambda i: (i, 0)),
            pl.BlockSpec((C, C), lambda i: (0, 0)),
            pl.BlockSpec((1, C), lambda i: (0, 0)),
        ],
        out_specs=pl.BlockSpec((block, C), lambda i: (i, 0)),
        out_shape=jax.ShapeDtypeStruct((mp, C), jnp.float32),
    )(xp, w, b.reshape(1, C))
    return out[:m] if pad else out


def _bn(h, g, b):
    mu = jnp.mean(h, axis=0, keepdims=True)
    var = jnp.var(h, axis=0, keepdims=True)
    return (h - mu) * jax.lax.rsqrt(var + 1e-5) * g + b


def _conv(elist, ei1, ei2, ei3, tri_list, invs, W, b, g, bt):
    agg = [jnp.zeros_like(e) for e in elist]
    e0 = elist[0]
    agg[1] = agg[1] + e0[ei1[0]] + e0[ei1[1]]
    agg[2] = agg[2] + e0[ei2[0]] + e0[ei2[1]]
    agg[3] = agg[3] + e0[ei3[0]] + e0[ei3[1]]
    agg[0] = agg[0].at[ei1[0]].add(elist[1]).at[ei1[1]].add(elist[1])
    for t, (da, db, dc) in tri_list:
        ia, ib, ic = t[0], t[1], t[2]
        A, B, Cv = elist[da][ia], elist[db][ib], elist[dc][ic]
        agg[da] = agg[da].at[ia].add(B * Cv)
        agg[db] = agg[db].at[ib].add(A * Cv)
        agg[dc] = agg[dc].at[ic].add(A * B)
    out = []
    for d in range(4):
        h = (1.0 + EPS) * elist[d] + agg[d]
        h = _matmul_bias(h, W[d], b[d])
        h = jax.nn.relu(_bn(h, g[d], bt[d]))
        out.append(h)
    for d, inv in zip((1, 2, 3), invs):
        out[d] = 0.5 * (out[d] + out[d][inv])
    return out


def kernel(edge_attr0, edge_attr1, edge_attr2, edge_attr3, edge_index0, edge_index, edge_index2, edge_index3, triangle_0_1_1, triangle_1_1_1, triangle_1_1_2, triangle_1_2_2, triangle_2_2_2, triangle_3_2_1, triangle_3_3_1, inverse_edge_1, inverse_edge_2, inverse_edge_3, Wagg, bagg, gamma, beta, Wout, bout):
    tri_list = list(zip([triangle_0_1_1, triangle_1_1_1, triangle_1_1_2, triangle_1_2_2, triangle_2_2_2, triangle_3_2_1, triangle_3_3_1], TRI_TYPES))
    invs = (inverse_edge_1, inverse_edge_2, inverse_edge_3)
    elist = [edge_attr0, edge_attr1, edge_attr2, edge_attr3]
    for l in range(L):
        elist = _conv(elist, edge_index, edge_index2, edge_index3, tri_list, invs, Wagg[l], bagg[l], gamma[l], beta[l])
    return tuple(_matmul_bias(h, Wout, bout) for h in elist)
